# Initial kernel scaffold; baseline (speedup 1.0000x reference)
import jax
import jax.numpy as jnp
from jax.experimental import pallas as pl


def kernel(input_ids):
    raise NotImplementedError("write your pallas kernel here")


# speedup vs baseline: 1.1046x; 1.1046x over previous
import functools

import jax
jax.config.update('jax_enable_x64', True)
import jax.numpy as jnp
import numpy as np
from jax import lax
from jax.experimental import pallas as pl
from jax.experimental.pallas import tpu as pltpu
from jax.experimental.pallas import tpu_sc as plsc


def _prime_check(n):
    if n < 2:
        return False
    if n % 2 == 0:
        return n == 2
    i = 3
    while i * i <= n:
        if n % i == 0:
            return False
        i += 2
    return True


def _head_primes():
    seen = set()
    primes = []
    for _ in range(2):
        s = 200000 - 1
        for _ in range(8):
            c = s + 1
            while not (_prime_check(c) and c not in seen):
                c += 1
            seen.add(c)
            primes.append(c)
            s = c
    return primes


_PRIMES = _head_primes()
_INVS = [float(np.float32(1.0) / np.float32(p)) for p in _PRIMES]

_B, _S = 4, 8192
_NW = 32
_CHUNK = (_B * _S) // _NW
_STEPS = _CHUNK // 16
_ROW_CHUNKS = _S // _CHUNK


@functools.cache
def _build_hash_kernel():
    return pl.kernel(
        _hash_body,
        out_type=jax.ShapeDtypeStruct((_B * _S * 16,), jnp.int32),
        mesh=plsc.VectorSubcoreMesh(core_axis_name="c", subcore_axis_name="s"),
        scratch_types=[
            pltpu.VMEM((_CHUNK + 16,), jnp.int32),
            pltpu.VMEM((_CHUNK * 16,), jnp.int32),
        ],
        compiler_params=pltpu.CompilerParams(needs_layout_passes=False),
    )


def _hash_body(x_hbm, out_hbm, xbuf, obuf):
    wid = lax.axis_index("s") * 2 + lax.axis_index("c")
    base = wid * _CHUNK
    xbuf[pl.ds(0, 16)] = jnp.zeros((16,), jnp.int32)
    pltpu.sync_copy(x_hbm.at[pl.ds(base, _CHUNK)], xbuf.at[pl.ds(16, _CHUNK)])

    @pl.when((wid & (_ROW_CHUNKS - 1)) > 0)
    def _():
        pltpu.sync_copy(x_hbm.at[pl.ds(base - 8, 8)], xbuf.at[pl.ds(8, 8)])

    lane16 = lax.iota(jnp.int32, 16) * 16

    def step(i, carry):
        off = 16 + i * 16
        x0 = xbuf[pl.ds(off, 16)]
        x1 = xbuf[pl.ds(off - 1, 16)]
        x2 = xbuf[pl.ds(off - 2, 16)]
        m1 = (x0 * 3) ^ (x1 * 5)
        m2 = m1 ^ (x2 * 7)
        f1 = m1.astype(jnp.float32)
        f2 = m2.astype(jnp.float32)
        ibase = i * 256 + lane16
        for h in range(16):
            m = m1 if h < 8 else m2
            f = f1 if h < 8 else f2
            p = jnp.int32(_PRIMES[h])
            q = (f * jnp.float32(_INVS[h])).astype(jnp.int32)
            r = m - q * p
            r = jnp.where(r < 0, r + p, r)
            r = jnp.where(r >= p, r - p, r)
            plsc.store_scatter(obuf, [ibase + h], r)
        return carry

    lax.fori_loop(jnp.int32(0), jnp.int32(_STEPS), step, jnp.int32(0))
    pltpu.sync_copy(obuf, out_hbm.at[pl.ds(base * 16, _CHUNK * 16)])


def kernel(input_ids):
    B, S = input_ids.shape
    assert (B, S) == (_B, _S)
    x32 = input_ids.astype(jnp.int32).reshape(-1)
    out = _build_hash_kernel()(x32)
    return out.reshape(B, S, 16).astype(input_ids.dtype)


# speedup vs baseline: 12.4426x; 11.2645x over previous
import functools

import jax
jax.config.update('jax_enable_x64', True)
import jax.numpy as jnp
import numpy as np
from jax import lax
from jax.experimental import pallas as pl
from jax.experimental.pallas import tpu as pltpu
from jax.experimental.pallas import tpu_sc as plsc


def _prime_check(n):
    if n < 2:
        return False
    if n % 2 == 0:
        return n == 2
    i = 3
    while i * i <= n:
        if n % i == 0:
            return False
        i += 2
    return True


def _head_primes():
    seen = set()
    primes = []
    for _ in range(2):
        s = 200000 - 1
        for _ in range(8):
            c = s + 1
            while not (_prime_check(c) and c not in seen):
                c += 1
            seen.add(c)
            primes.append(c)
            s = c
    return primes


_PRIMES = _head_primes()
_INVS = [float(np.float32(1.0) / np.float32(p)) for p in _PRIMES]

_B, _S = 4, 8192
_NW = 32
_CHUNK = (_B * _S) // _NW
_STEPS = _CHUNK // 16
_ROW_CHUNKS = _S // _CHUNK


@functools.cache
def _build_hash_kernel():
    return pl.kernel(
        _hash_body,
        out_type=jax.ShapeDtypeStruct((_B, 16, _S), jnp.int32),
        mesh=plsc.VectorSubcoreMesh(core_axis_name="c", subcore_axis_name="s"),
        scratch_types=[
            pltpu.VMEM((_CHUNK + 16,), jnp.int32),
            pltpu.VMEM((16, _CHUNK), jnp.int32),
        ],
        compiler_params=pltpu.CompilerParams(needs_layout_passes=False),
    )


def _hash_body(x_hbm, out_hbm, xbuf, obuf):
    wid = lax.axis_index("s") * 2 + lax.axis_index("c")
    base = wid * _CHUNK
    row = wid // _ROW_CHUNKS
    col = wid & (_ROW_CHUNKS - 1)
    xbuf[pl.ds(0, 16)] = jnp.zeros((16,), jnp.int32)
    pltpu.sync_copy(x_hbm.at[pl.ds(base, _CHUNK)], xbuf.at[pl.ds(16, _CHUNK)])

    @pl.when(col > 0)
    def _():
        pltpu.sync_copy(x_hbm.at[pl.ds(base - 8, 8)], xbuf.at[pl.ds(8, 8)])

    def step(i, carry):
        off = 16 + i * 16
        x0 = xbuf[pl.ds(off, 16)]
        x1 = xbuf[pl.ds(off - 1, 16)]
        x2 = xbuf[pl.ds(off - 2, 16)]
        m1 = (x0 * 3) ^ (x1 * 5)
        m2 = m1 ^ (x2 * 7)
        f1 = m1.astype(jnp.float32)
        f2 = m2.astype(jnp.float32)
        for h in range(16):
            m = m1 if h < 8 else m2
            f = f1 if h < 8 else f2
            p = jnp.int32(_PRIMES[h])
            q = (f * jnp.float32(_INVS[h])).astype(jnp.int32)
            r = m - q * p
            r = jnp.where(r < 0, r + p, r)
            r = jnp.where(r >= p, r - p, r)
            obuf[h, pl.ds(i * 16, 16)] = r
        return carry

    lax.fori_loop(jnp.int32(0), jnp.int32(_STEPS), step, jnp.int32(0))
    pltpu.sync_copy(obuf, out_hbm.at[row, :, pl.ds(col * _CHUNK, _CHUNK)])


def kernel(input_ids):
    B, S = input_ids.shape
    assert (B, S) == (_B, _S)
    x32 = input_ids.astype(jnp.int32).reshape(-1)
    out = _build_hash_kernel()(x32)
    return out.astype(input_ids.dtype).transpose(0, 2, 1)


# speedup vs baseline: 12.8052x; 1.0291x over previous
import functools

import jax
jax.config.update('jax_enable_x64', True)
import jax.numpy as jnp
import numpy as np
from jax import lax
from jax.experimental import pallas as pl
from jax.experimental.pallas import tpu as pltpu
from jax.experimental.pallas import tpu_sc as plsc


def _prime_check(n):
    if n < 2:
        return False
    if n % 2 == 0:
        return n == 2
    i = 3
    while i * i <= n:
        if n % i == 0:
            return False
        i += 2
    return True


def _head_primes():
    seen = set()
    primes = []
    for _ in range(2):
        s = 200000 - 1
        for _ in range(8):
            c = s + 1
            while not (_prime_check(c) and c not in seen):
                c += 1
            seen.add(c)
            primes.append(c)
            s = c
    return primes


_PRIMES = _head_primes()
_INVS = [float(np.nextafter(np.float32(1.0) / np.float32(p), np.float32(0.0)))
         for p in _PRIMES]

_B, _S = 4, 8192
_NW = 32
_CHUNK = (_B * _S) // _NW
_STEPS = _CHUNK // 16
_ROW_CHUNKS = _S // _CHUNK


@functools.cache
def _build_hash_kernel():
    return pl.kernel(
        _hash_body,
        out_type=jax.ShapeDtypeStruct((_B, 16, _S), jnp.int32),
        mesh=plsc.VectorSubcoreMesh(core_axis_name="c", subcore_axis_name="s"),
        scratch_types=[
            pltpu.VMEM((_CHUNK + 16,), jnp.int32),
            pltpu.VMEM((16, _CHUNK), jnp.int32),
        ],
        compiler_params=pltpu.CompilerParams(needs_layout_passes=False),
    )


def _hash_body(x_hbm, out_hbm, xbuf, obuf):
    wid = lax.axis_index("s") * 2 + lax.axis_index("c")
    base = wid * _CHUNK
    row = wid // _ROW_CHUNKS
    col = wid & (_ROW_CHUNKS - 1)
    xbuf[pl.ds(0, 16)] = jnp.zeros((16,), jnp.int32)
    pltpu.sync_copy(x_hbm.at[pl.ds(base, _CHUNK)], xbuf.at[pl.ds(16, _CHUNK)])

    @pl.when(col > 0)
    def _():
        pltpu.sync_copy(x_hbm.at[pl.ds(base - 8, 8)], xbuf.at[pl.ds(8, 8)])

    def step(i, carry):
        off = 16 + i * 16
        x0 = xbuf[pl.ds(off, 16)]
        x1 = xbuf[pl.ds(off - 1, 16)]
        x2 = xbuf[pl.ds(off - 2, 16)]
        m1 = (x0 * 3) ^ (x1 * 5)
        m2 = m1 ^ (x2 * 7)
        f1 = m1.astype(jnp.float32)
        f2 = m2.astype(jnp.float32)
        for h in range(16):
            m = m1 if h < 8 else m2
            f = f1 if h < 8 else f2
            p = jnp.int32(_PRIMES[h])
            q = (f * jnp.float32(_INVS[h])).astype(jnp.int32)
            r = m - q * p
            r = jnp.where(r >= p, r - p, r)
            obuf[h, pl.ds(i * 16, 16)] = r
        return carry

    lax.fori_loop(jnp.int32(0), jnp.int32(_STEPS), step, jnp.int32(0))
    pltpu.sync_copy(obuf, out_hbm.at[row, :, pl.ds(col * _CHUNK, _CHUNK)])


def kernel(input_ids):
    B, S = input_ids.shape
    assert (B, S) == (_B, _S)
    x32 = input_ids.astype(jnp.int32).reshape(-1)
    out = _build_hash_kernel()(x32)
    return out.astype(input_ids.dtype).transpose(0, 2, 1)
